# baseline (speedup 1.0000x reference)
import functools

import jax
import jax.numpy as jnp
from jax import lax
from jax.experimental import pallas as pl
from jax.experimental.pallas import tpu as pltpu
from jax.experimental.pallas import tpu_sc as plsc

_NUM_CLASSES = 100000
_BATCH = 1024
_SMOOTH = 0.1
_NEG = _SMOOTH / (_NUM_CLASSES - 1)
_POS = 1.0 - _SMOOTH

_RTC = 256
_BR = 16
_GRID = _RTC // _BR
_GPR = _BATCH // _GRID

_NC = 2
_NS = 16
_NW = _NC * _NS

_SC_ROWS = _BATCH - _RTC
_RPW = _SC_ROWS // _NW
_CFULL = 99968
_CCHUNK = 6400
_CHUNKS = [(c, _CCHUNK) for c in range(0, 96000, _CCHUNK)] + [(96000, 3968)]

_NBUF = 4


def _dma_in(x_hbm, bufs, sems, blk, slot):
    return pltpu.make_async_copy(
        x_hbm.at[pl.ds(blk * _BR, _BR), :], bufs.at[slot], sems.at[slot]
    )


def _main_body(tgt_ref, x_hbm, tile_ref, tot_ref, acc_ref, bufs, sems, gsems):
    i = pl.program_id(0)

    @pl.when(i == 0)
    def _init():
        acc_ref[0] = 0.0
        for k in range(_NBUF):
            _dma_in(x_hbm, bufs, sems, k, k).start()

    for k in range(_GPR):
        row = i * _GPR + k
        t = tgt_ref[row]
        start = pl.multiple_of((t // 128) * 128, 128)
        pltpu.make_async_copy(
            x_hbm.at[pl.ds(row, 1), pl.ds(start, 128)],
            tile_ref.at[pl.ds(row, 1), :],
            gsems.at[k % 8],
        ).start()

    slot = lax.rem(i, _NBUF)
    _dma_in(x_hbm, bufs, sems, i, slot).wait()
    acc_ref[0] += jnp.sum(bufs[slot])

    @pl.when(i + _NBUF < _GRID)
    def _next():
        _dma_in(x_hbm, bufs, sems, i + _NBUF, slot).start()

    @pl.when(i == _GRID - 1)
    def _fini():
        for q in range(8):
            pltpu.make_async_copy(
                x_hbm.at[pl.ds(0, _BATCH // 8), pl.ds(0, 128)],
                tile_ref.at[pl.ds(0, _BATCH // 8), :],
                gsems.at[q],
            ).wait()
        tot_ref[0] = acc_ref[0]


_main = pl.pallas_call(
    _main_body,
    grid=(_GRID,),
    in_specs=[
        pl.BlockSpec(memory_space=pltpu.SMEM),
        pl.BlockSpec(memory_space=pl.ANY),
    ],
    out_specs=[
        pl.BlockSpec(memory_space=pl.ANY),
        pl.BlockSpec(memory_space=pltpu.SMEM),
    ],
    out_shape=[
        jax.ShapeDtypeStruct((_BATCH, 128), jnp.float32),
        jax.ShapeDtypeStruct((1,), jnp.float32),
    ],
    scratch_shapes=[
        pltpu.SMEM((1,), jnp.float32),
        pltpu.VMEM((_NBUF, _BR, _NUM_CLASSES), jnp.float32),
        pltpu.SemaphoreType.DMA((_NBUF,)),
        pltpu.SemaphoreType.DMA((8,)),
    ],
    compiler_params=pltpu.CompilerParams(dimension_semantics=("arbitrary",)),
)


def _make_scsum():
    mesh = plsc.VectorSubcoreMesh(core_axis_name="c", subcore_axis_name="s")

    @functools.partial(
        pl.kernel,
        mesh=mesh,
        out_type=jax.ShapeDtypeStruct((_NW * 16,), jnp.float32),
        scratch_types=[
            pltpu.VMEM((8, _CCHUNK), jnp.float32),
            pltpu.VMEM((8, _CCHUNK), jnp.float32),
            pltpu.VMEM((16,), jnp.float32),
            pltpu.SemaphoreType.DMA,
            pltpu.SemaphoreType.DMA,
        ],
    )
    def scsum_kernel(x_hbm, out_hbm, buf0, buf1, val_v, sem0, sem1):
        wid = lax.axis_index("s") * _NC + lax.axis_index("c")
        row0 = _RTC + wid * _RPW
        bufs = [buf0, buf1]
        sems = [sem0, sem1]
        work = [
            (g * 8, c0, w)
            for g in range(_RPW // 8)
            for (c0, w) in _CHUNKS
        ]

        def _copy(idx):
            r, c0, w = work[idx]
            b = bufs[idx % 2]
            return pltpu.make_async_copy(
                x_hbm.at[pl.ds(row0 + r, 8), pl.ds(c0, w)],
                b.at[:, pl.ds(0, w)],
                sems[idx % 2],
            )

        _copy(0).start()
        acc = jnp.zeros((16,), jnp.float32)
        for idx in range(len(work)):
            if idx + 1 < len(work):
                _copy(idx + 1).start()
            _copy(idx).wait()
            b = bufs[idx % 2]
            w = work[idx][2]

            def body(o, a):
                for r in range(8):
                    a = a + b[r, pl.ds(o * 16, 16)]
                return a

            acc = lax.fori_loop(0, w // 16, body, acc)
        val_v[...] = acc
        pltpu.sync_copy(val_v, out_hbm.at[pl.ds(wid * 16, 16)])

    return scsum_kernel


_scsum = _make_scsum()


def _make_pick():
    mesh = plsc.VectorSubcoreMesh(core_axis_name="c", subcore_axis_name="s")
    bpw = _BATCH // _NW

    @functools.partial(
        pl.kernel,
        mesh=mesh,
        out_type=jax.ShapeDtypeStruct((_BATCH,), jnp.float32),
        scratch_types=[
            pltpu.VMEM((bpw,), jnp.int32),
            pltpu.VMEM((bpw, 128), jnp.float32),
            pltpu.VMEM((bpw,), jnp.float32),
        ],
    )
    def pick_kernel(y_hbm, tgt_hbm, out_hbm, col_v, buf_v, val_v):
        wid = lax.axis_index("s") * _NC + lax.axis_index("c")
        base = wid * bpw
        pltpu.sync_copy(tgt_hbm.at[pl.ds(base, bpw)], col_v)
        pltpu.sync_copy(y_hbm.at[pl.ds(base, bpw), :], buf_v)
        lane_iota = lax.iota(jnp.int32, 16)
        for j in range(bpw // 16):
            sl = pl.ds(j * 16, 16)
            cols16 = col_v[sl]
            lanes16 = cols16 % 16
            code16 = lane_iota * 128 + ((cols16 % 128) - lanes16)
            val16 = jnp.zeros((16,), jnp.float32)
            for k in range(16):
                i = j * 16 + k
                for s in range(8):
                    seg = buf_v[i, pl.ds(s * 16, 16)]
                    g = seg[lanes16]
                    val16 = jnp.where(code16 == (k * 128 + s * 16), g, val16)
            val_v[sl] = val16
        pltpu.sync_copy(val_v, out_hbm.at[pl.ds(base, bpw)])

    return pick_kernel


_pick = _make_pick()


def _combine_body(tot_ref, part_ref, p_ref, x_hbm, out_ref, strip_v, ssem):
    pltpu.make_async_copy(
        x_hbm.at[pl.ds(_RTC, _SC_ROWS), pl.ds(_CFULL, 32)], strip_v, ssem
    ).start()
    pltpu.make_async_copy(
        x_hbm.at[pl.ds(_RTC, _SC_ROWS), pl.ds(_CFULL, 32)], strip_v, ssem
    ).wait()
    total = tot_ref[0] + jnp.sum(part_ref[...]) + jnp.sum(strip_v[...])
    g = jnp.sum(p_ref[...])
    out_ref[0] = -(_NEG * total + (_POS - _NEG) * g) / _BATCH


_combine = pl.pallas_call(
    _combine_body,
    in_specs=[
        pl.BlockSpec(memory_space=pltpu.SMEM),
        pl.BlockSpec((4, 128), lambda: (0, 0)),
        pl.BlockSpec((8, 128), lambda: (0, 0)),
        pl.BlockSpec(memory_space=pl.ANY),
    ],
    out_specs=pl.BlockSpec(memory_space=pltpu.SMEM),
    out_shape=jax.ShapeDtypeStruct((1,), jnp.float32),
    scratch_shapes=[
        pltpu.VMEM((_SC_ROWS, 32), jnp.float32),
        pltpu.SemaphoreType.DMA,
    ],
)


def kernel(log_softmax, target):
    tgt = target.astype(jnp.int32)
    parts = _scsum(log_softmax)
    tiles, total = _main(tgt, log_softmax)
    picked = _pick(tiles, tgt)
    out = _combine(
        total, parts.reshape(4, 128), picked.reshape(8, 128), log_softmax
    )
    return out[0]


# speedup vs baseline: 1.0108x; 1.0108x over previous
import functools

import jax
import jax.numpy as jnp
from jax import lax
from jax.experimental import pallas as pl
from jax.experimental.pallas import tpu as pltpu
from jax.experimental.pallas import tpu_sc as plsc

_NUM_CLASSES = 100000
_BATCH = 1024
_SMOOTH = 0.1
_NEG = _SMOOTH / (_NUM_CLASSES - 1)
_POS = 1.0 - _SMOOTH

_NC = 2
_NS = 16
_NW = _NC * _NS

_CK = 74240
_RPW = _CK // _NW
_TCR = _NUM_CLASSES - _CK
_BR = 368
_GRID = _TCR // _BR
_GPR = -(-_BATCH // _GRID)

_NBUF = 4


def _dma_in(xt_hbm, bufs, sems, blk, slot):
    return pltpu.make_async_copy(
        xt_hbm.at[pl.ds(_CK + blk * _BR, _BR), :], bufs.at[slot], sems.at[slot]
    )


def _main_body(tgt_ref, xt_hbm, yt_ref, tot_ref, acc_ref, bufs, sems, gsem):
    i = pl.program_id(0)

    @pl.when(i == 0)
    def _init():
        acc_ref[0] = 0.0
        for k in range(_NBUF):
            _dma_in(xt_hbm, bufs, sems, k, k).start()

    for k in range(_GPR):
        row = i * _GPR + k

        @pl.when(row < _BATCH)
        def _gather(row=row):
            t = tgt_ref[row]
            c8 = pl.multiple_of((t // 8) * 8, 8)
            pltpu.make_async_copy(
                xt_hbm.at[pl.ds(c8, 8), pl.ds((row // 128) * 128, 128)],
                yt_ref.at[pl.ds(row * 8, 8), :],
                gsem,
            ).start()

    slot = lax.rem(i, _NBUF)
    _dma_in(xt_hbm, bufs, sems, i, slot).wait()
    acc_ref[0] += jnp.sum(bufs[slot])

    @pl.when(i + _NBUF < _GRID)
    def _next():
        _dma_in(xt_hbm, bufs, sems, i + _NBUF, slot).start()

    @pl.when(i == _GRID - 1)
    def _fini():
        pltpu.make_async_copy(
            xt_hbm.at[pl.ds(0, _BATCH * 8), pl.ds(0, 128)], yt_ref, gsem
        ).wait()
        tot_ref[0] = acc_ref[0]


_main = pl.pallas_call(
    _main_body,
    grid=(_GRID,),
    in_specs=[
        pl.BlockSpec(memory_space=pltpu.SMEM),
        pl.BlockSpec(memory_space=pl.ANY),
    ],
    out_specs=[
        pl.BlockSpec(memory_space=pl.ANY),
        pl.BlockSpec(memory_space=pltpu.SMEM),
    ],
    out_shape=[
        jax.ShapeDtypeStruct((_BATCH * 8, 128), jnp.float32),
        jax.ShapeDtypeStruct((1,), jnp.float32),
    ],
    scratch_shapes=[
        pltpu.SMEM((1,), jnp.float32),
        pltpu.VMEM((_NBUF, _BR, _BATCH), jnp.float32),
        pltpu.SemaphoreType.DMA((_NBUF,)),
        pltpu.SemaphoreType.DMA,
    ],
    compiler_params=pltpu.CompilerParams(dimension_semantics=("arbitrary",)),
)


def _make_scsum():
    mesh = plsc.VectorSubcoreMesh(core_axis_name="c", subcore_axis_name="s")

    @functools.partial(
        pl.kernel,
        mesh=mesh,
        out_type=jax.ShapeDtypeStruct((_NW * 16,), jnp.float32),
        scratch_types=[
            pltpu.VMEM((8, _BATCH), jnp.float32),
            pltpu.VMEM((16,), jnp.float32),
        ],
    )
    def scsum_kernel(xt_hbm, out_hbm, buf, val_v):
        wid = lax.axis_index("s") * _NC + lax.axis_index("c")
        row0 = wid * _RPW

        def chunk(j, acc):
            r = pl.multiple_of(row0 + j * 8, 8)
            pltpu.sync_copy(xt_hbm.at[pl.ds(r, 8), :], buf)

            def body(o, a):
                for rr in range(8):
                    a = a + buf[rr, pl.ds(o * 16, 16)]
                return a

            return lax.fori_loop(0, _BATCH // 16, body, acc)

        acc = lax.fori_loop(0, _RPW // 8, chunk, jnp.zeros((16,), jnp.float32))
        val_v[...] = acc
        pltpu.sync_copy(val_v, out_hbm.at[pl.ds(wid * 16, 16)])

    return scsum_kernel


_scsum = _make_scsum()


def _make_pick():
    mesh = plsc.VectorSubcoreMesh(core_axis_name="c", subcore_axis_name="s")
    bpw = _BATCH // _NW

    @functools.partial(
        pl.kernel,
        mesh=mesh,
        out_type=jax.ShapeDtypeStruct((_BATCH,), jnp.float32),
        scratch_types=[
            pltpu.VMEM((bpw,), jnp.int32),
            pltpu.VMEM((bpw * 8, 128), jnp.float32),
            pltpu.VMEM((bpw,), jnp.float32),
        ],
    )
    def pick_kernel(yt_hbm, tgt_hbm, out_hbm, col_v, buf_v, val_v):
        wid = lax.axis_index("s") * _NC + lax.axis_index("c")
        base = wid * bpw
        pltpu.sync_copy(tgt_hbm.at[pl.ds(base, bpw)], col_v)
        pltpu.sync_copy(yt_hbm.at[pl.ds(base * 8, bpw * 8), :], buf_v)
        lane_iota = lax.iota(jnp.int32, 16)
        for j in range(bpw // 16):
            sl = pl.ds(j * 16, 16)
            cols16 = col_v[sl]
            code16 = lane_iota * 8 + (cols16 % 8)
            val16 = jnp.zeros((16,), jnp.float32)
            for k in range(16):
                i = j * 16 + k
                r = base + i
                li = (r % 128) % 16
                sgi = (r % 128) // 16
                bc16 = lane_iota * 0 + li
                for s in range(8):
                    seg = buf_v[i * 8 + s, pl.ds(sgi * 16, 16)]
                    g = seg[bc16]
                    val16 = jnp.where(code16 == (k * 8 + s), g, val16)
            val_v[sl] = val16
        pltpu.sync_copy(val_v, out_hbm.at[pl.ds(base, bpw)])

    return pick_kernel


_pick = _make_pick()


def _combine_body(tot_ref, part_ref, p_ref, out_ref):
    total = tot_ref[0] + jnp.sum(part_ref[...])
    g = jnp.sum(p_ref[...])
    out_ref[0] = -(_NEG * total + (_POS - _NEG) * g) / _BATCH


_combine = pl.pallas_call(
    _combine_body,
    in_specs=[
        pl.BlockSpec(memory_space=pltpu.SMEM),
        pl.BlockSpec((4, 128), lambda: (0, 0)),
        pl.BlockSpec((8, 128), lambda: (0, 0)),
    ],
    out_specs=pl.BlockSpec(memory_space=pltpu.SMEM),
    out_shape=jax.ShapeDtypeStruct((1,), jnp.float32),
)


def kernel(log_softmax, target):
    tgt = target.astype(jnp.int32)
    xt = log_softmax.T
    parts = _scsum(xt)
    tiles, total = _main(tgt, xt)
    picked = _pick(tiles, tgt)
    out = _combine(total, parts.reshape(4, 128), picked.reshape(8, 128))
    return out[0]
